# baseline (speedup 1.0000x reference)
import functools

import jax
import jax.numpy as jnp
from jax import lax
from jax.experimental import pallas as pl
from jax.experimental.pallas import tpu as pltpu
from jax.experimental.pallas import tpu_sc as plsc

N = 10000
E = 320000
D = 128
R = 8

K = 128
NC = 2
NS = 16
NW = NC * NS
CPW = 80
EPAD = NW * CPW * K
NPAD = 10240
ROWS_PER_TILE = NPAD // NS


def _ymm_body(x_ref, w_ref, y_ref):
    y_ref[0] = jnp.dot(x_ref[...], w_ref[0], preferred_element_type=jnp.float32)


def _relation_matmul(x, W):
    BN = 2000
    return pl.pallas_call(
        _ymm_body,
        grid=(R, N // BN),
        in_specs=[
            pl.BlockSpec((BN, D), lambda r, b: (b, 0)),
            pl.BlockSpec((1, D, D), lambda r, b: (r, 0, 0)),
        ],
        out_specs=pl.BlockSpec((1, BN, D), lambda r, b: (r, b, 0)),
        out_shape=jax.ShapeDtypeStruct((R, N, D), jnp.float32),
    )(x, W)


def _make_sc_segment_sum():
    mesh = plsc.VectorSubcoreMesh(core_axis_name="c", subcore_axis_name="s")

    @functools.partial(
        pl.kernel,
        out_type=jax.ShapeDtypeStruct((NC, NPAD, D), jnp.float32),
        mesh=mesh,
        scratch_types=[
            pltpu.VMEM((2, K), jnp.int32),
            pltpu.VMEM((K, 16), jnp.float32),
            pltpu.VMEM((K, D), jnp.float32),
            pltpu.VMEM_SHARED((NPAD, D), jnp.float32),
        ],
    )
    def sc_segment_sum(y_hbm, meta_hbm, ewx_hbm, out_hbm,
                       meta_v, ewx_v, rows_v, acc):
        cid = lax.axis_index("c")
        sid = lax.axis_index("s")
        wid = sid * NC + cid

        def zrow(j, _):
            for i in range(D // 16):
                rows_v[j, pl.ds(i * 16, 16)] = jnp.zeros((16,), jnp.float32)
            return _
        lax.fori_loop(0, K, zrow, None)
        base = sid * ROWS_PER_TILE
        for t in range(ROWS_PER_TILE // K):
            pltpu.sync_copy(rows_v, acc.at[pl.ds(base + t * K, K)])
        plsc.subcore_barrier()

        def chunk_body(j, _):
            c = wid + j * NW
            pltpu.sync_copy(meta_hbm.at[c], meta_v)
            pltpu.sync_copy(ewx_hbm.at[c], ewx_v)
            pltpu.sync_copy(y_hbm.at[meta_v.at[0]], rows_v)

            def row_body(jb, carry):
                for q in range(16):
                    jj = jb * 16 + q
                    w = ewx_v[jj]
                    for i in range(D // 16):
                        rows_v[jj, pl.ds(i * 16, 16)] = (
                            rows_v[jj, pl.ds(i * 16, 16)] * w)
                return carry
            lax.fori_loop(0, K // 16, row_body, None)

            pltpu.sync_copy(rows_v, acc.at[meta_v.at[1]], add=True)
            return _
        lax.fori_loop(0, CPW, chunk_body, None)
        plsc.subcore_barrier()

        pltpu.sync_copy(acc.at[pl.ds(base, ROWS_PER_TILE)],
                        out_hbm.at[cid, pl.ds(base, ROWS_PER_TILE)])

    return sc_segment_sum


_sc_segment_sum = _make_sc_segment_sum()


def _epilogue_body(p_ref, x_ref, o_ref):
    m = p_ref[0] + p_ref[1]
    n1 = jnp.sqrt(jnp.sum(m * m, axis=1, keepdims=True))
    m = m / jnp.maximum(n1, 1e-6)
    h = m + x_ref[...]
    n2 = jnp.sqrt(jnp.sum(h * h, axis=1, keepdims=True))
    o_ref[...] = h / n2


def _epilogue(partials, x):
    BN = 2000
    return pl.pallas_call(
        _epilogue_body,
        grid=(N // BN,),
        in_specs=[
            pl.BlockSpec((NC, BN, D), lambda b: (0, b, 0)),
            pl.BlockSpec((BN, D), lambda b: (b, 0)),
        ],
        out_specs=pl.BlockSpec((BN, D), lambda b: (b, 0)),
        out_shape=jax.ShapeDtypeStruct((N, D), jnp.float32),
    )(partials, x)


def kernel(x, edge_index, edge_type, edge_weight, W):
    src = edge_index[0]
    dst = edge_index[1]
    gidx = edge_type * N + src
    pad = EPAD - E
    meta = jnp.stack([gidx, dst])
    meta = jnp.pad(meta, ((0, 0), (0, pad)))
    meta = meta.reshape(2, CPW * NW, K).transpose(1, 0, 2)
    ewx = jnp.broadcast_to(
        jnp.pad(edge_weight, (0, pad)).reshape(CPW * NW, K, 1),
        (CPW * NW, K, 16)).astype(jnp.float32)

    y = _relation_matmul(x, W).reshape(R * N, D)
    partials = _sc_segment_sum(y, meta, ewx)
    return _epilogue(partials, x)


# speedup vs baseline: 1.1761x; 1.1761x over previous
import functools

import jax
import jax.numpy as jnp
from jax import lax
from jax.experimental import pallas as pl
from jax.experimental.pallas import tpu as pltpu
from jax.experimental.pallas import tpu_sc as plsc

N = 10000
E = 320000
D = 128
R = 8

K = 128
NC = 2
NS = 16
NW = NC * NS
CPW = 80
EPAD = NW * CPW * K
NPAD = 10240
ROWS_PER_TILE = NPAD // NS


def _ymm_body(x_ref, w_ref, y_ref):
    y_ref[0] = jnp.dot(x_ref[...], w_ref[0], preferred_element_type=jnp.float32)


def _relation_matmul(x, W):
    BN = 2000
    return pl.pallas_call(
        _ymm_body,
        grid=(R, N // BN),
        in_specs=[
            pl.BlockSpec((BN, D), lambda r, b: (b, 0)),
            pl.BlockSpec((1, D, D), lambda r, b: (r, 0, 0)),
        ],
        out_specs=pl.BlockSpec((1, BN, D), lambda r, b: (r, b, 0)),
        out_shape=jax.ShapeDtypeStruct((R, N, D), jnp.float32),
    )(x, W)


def _make_sc_segment_sum():
    mesh = plsc.VectorSubcoreMesh(core_axis_name="c", subcore_axis_name="s")

    @functools.partial(
        pl.kernel,
        out_type=jax.ShapeDtypeStruct((NC, NPAD, D), jnp.float32),
        mesh=mesh,
        scratch_types=[
            pltpu.VMEM((2, K), jnp.int32),
            pltpu.VMEM((K,), jnp.float32),
            pltpu.VMEM((K, D), jnp.float32),
            pltpu.VMEM_SHARED((NPAD, D), jnp.float32),
            pltpu.SemaphoreType.DMA,
        ],
    )
    def sc_segment_sum(y_hbm, meta_hbm, ew_hbm, out_hbm,
                       meta_v, ew_v, rows_v, acc, sem):
        cid = lax.axis_index("c")
        sid = lax.axis_index("s")
        wid = sid * NC + cid

        def zrow(j, _):
            for i in range(D // 16):
                rows_v[j, pl.ds(i * 16, 16)] = jnp.zeros((16,), jnp.float32)
            return _
        lax.fori_loop(0, K, zrow, None)
        base = sid * ROWS_PER_TILE
        for t in range(ROWS_PER_TILE // K):
            pltpu.sync_copy(rows_v, acc.at[pl.ds(base + t * K, K)])
        plsc.subcore_barrier()

        def chunk_body(j, _):
            c = wid + j * NW
            pltpu.sync_copy(meta_hbm.at[c], meta_v)
            pltpu.sync_copy(ew_hbm.at[c], ew_v)
            pltpu.async_copy(y_hbm.at[meta_v.at[0]], rows_v, sem).wait()

            def row_body(jb, carry):
                ew16 = ew_v[pl.ds(jb * 16, 16)]
                for q in range(16):
                    w = ew16[q]
                    jj = jb * 16 + q
                    for i in range(D // 16):
                        rows_v[jj, pl.ds(i * 16, 16)] = (
                            rows_v[jj, pl.ds(i * 16, 16)] * w)
                return carry
            lax.fori_loop(0, K // 16, row_body, None)

            pltpu.sync_copy(rows_v, acc.at[meta_v.at[1]], add=True)
            return _
        lax.fori_loop(0, CPW, chunk_body, None)
        plsc.subcore_barrier()

        pltpu.sync_copy(acc.at[pl.ds(base, ROWS_PER_TILE)],
                        out_hbm.at[cid, pl.ds(base, ROWS_PER_TILE)])

    return sc_segment_sum


_sc_segment_sum = _make_sc_segment_sum()


def _epilogue_body(p_ref, x_ref, o_ref):
    m = p_ref[0] + p_ref[1]
    n1 = jnp.sqrt(jnp.sum(m * m, axis=1, keepdims=True))
    m = m / jnp.maximum(n1, 1e-6)
    h = m + x_ref[...]
    n2 = jnp.sqrt(jnp.sum(h * h, axis=1, keepdims=True))
    o_ref[...] = h / n2


def _epilogue(partials, x):
    BN = 2000
    return pl.pallas_call(
        _epilogue_body,
        grid=(N // BN,),
        in_specs=[
            pl.BlockSpec((NC, BN, D), lambda b: (0, b, 0)),
            pl.BlockSpec((BN, D), lambda b: (b, 0)),
        ],
        out_specs=pl.BlockSpec((BN, D), lambda b: (b, 0)),
        out_shape=jax.ShapeDtypeStruct((N, D), jnp.float32),
    )(partials, x)


def kernel(x, edge_index, edge_type, edge_weight, W):
    src = edge_index[0]
    dst = edge_index[1]
    gidx = edge_type * N + src
    pad = EPAD - E
    meta = jnp.stack([gidx, dst])
    meta = jnp.pad(meta, ((0, 0), (0, pad)))
    meta = meta.reshape(2, CPW * NW, K).transpose(1, 0, 2)
    ew = jnp.pad(edge_weight, (0, pad)).reshape(CPW * NW, K)

    y = _relation_matmul(x, W).reshape(R * N, D)
    partials = _sc_segment_sum(y, meta, ew)
    return _epilogue(partials, x)


# speedup vs baseline: 1.9721x; 1.6768x over previous
import functools

import jax
import jax.numpy as jnp
from jax import lax
from jax.experimental import pallas as pl
from jax.experimental.pallas import tpu as pltpu
from jax.experimental.pallas import tpu_sc as plsc

N = 10000
E = 320000
D = 128
R = 8

K = 128
NC = 2
NS = 16
NW = NC * NS
CPW = 80
EPAD = NW * CPW * K
NPAD = 10240
ROWS_PER_TILE = NPAD // NS


def _ymm_body(x_ref, w_ref, y_ref):
    y_ref[0] = jnp.dot(x_ref[...], w_ref[0], preferred_element_type=jnp.float32)


def _relation_matmul(x, W):
    BN = 2000
    return pl.pallas_call(
        _ymm_body,
        grid=(R, N // BN),
        in_specs=[
            pl.BlockSpec((BN, D), lambda r, b: (b, 0)),
            pl.BlockSpec((1, D, D), lambda r, b: (r, 0, 0)),
        ],
        out_specs=pl.BlockSpec((1, BN, D), lambda r, b: (r, b, 0)),
        out_shape=jax.ShapeDtypeStruct((R, N, D), jnp.float32),
    )(x, W)


def _make_sc_segment_sum():
    mesh = plsc.VectorSubcoreMesh(core_axis_name="c", subcore_axis_name="s")

    @functools.partial(
        pl.kernel,
        out_type=jax.ShapeDtypeStruct((NC, NPAD, D), jnp.float32),
        mesh=mesh,
        scratch_types=[
            pltpu.VMEM((2, K), jnp.int32),
            pltpu.VMEM((K,), jnp.float32),
            pltpu.VMEM((K, D), jnp.float32),
            pltpu.VMEM_SHARED((NPAD, D), jnp.float32),
            pltpu.SemaphoreType.DMA,
        ],
    )
    def sc_segment_sum(y_hbm, meta_hbm, ew_hbm, out_hbm,
                       meta_v, ew_v, rows_v, acc, sem):
        cid = lax.axis_index("c")
        sid = lax.axis_index("s")
        wid = sid * NC + cid

        def zrow(j, _):
            for i in range(D // 16):
                rows_v[j, pl.ds(i * 16, 16)] = jnp.zeros((16,), jnp.float32)
            return _
        lax.fori_loop(0, K, zrow, None)
        base = sid * ROWS_PER_TILE
        for t in range(ROWS_PER_TILE // K):
            pltpu.sync_copy(rows_v, acc.at[pl.ds(base + t * K, K)])
        plsc.subcore_barrier()

        def chunk_body(j, _):
            c = wid + j * NW
            pltpu.sync_copy(meta_hbm.at[c], meta_v)
            pltpu.sync_copy(ew_hbm.at[c], ew_v)
            pltpu.async_copy(y_hbm.at[meta_v.at[0]], rows_v, sem).wait()

            def row_body(jb, carry):
                ew16 = ew_v[pl.ds(jb * 16, 16)]
                for q in range(16):
                    w = ew16[q]
                    jj = jb * 16 + q
                    for i in range(D // 16):
                        rows_v[jj, pl.ds(i * 16, 16)] = (
                            rows_v[jj, pl.ds(i * 16, 16)] * w)
                return carry
            lax.fori_loop(0, K // 16, row_body, None)

            pltpu.sync_copy(rows_v, acc.at[meta_v.at[1]], add=True)
            return _
        lax.fori_loop(0, CPW, chunk_body, None)
        plsc.subcore_barrier()

        pltpu.sync_copy(acc.at[pl.ds(base, ROWS_PER_TILE)],
                        out_hbm.at[cid, pl.ds(base, ROWS_PER_TILE)])

    return sc_segment_sum


_sc_segment_sum = _make_sc_segment_sum()


def _epilogue_body(p_ref, x_ref, o_ref):
    m = p_ref[0] + p_ref[1]
    n1 = jnp.sqrt(jnp.sum(m * m, axis=1, keepdims=True))
    m = m / jnp.maximum(n1, 1e-6)
    h = m + x_ref[...]
    n2 = jnp.sqrt(jnp.sum(h * h, axis=1, keepdims=True))
    o_ref[...] = h / n2


def _epilogue(partials, x):
    BN = 2000
    return pl.pallas_call(
        _epilogue_body,
        grid=(N // BN,),
        in_specs=[
            pl.BlockSpec((NC, BN, D), lambda b: (0, b, 0)),
            pl.BlockSpec((BN, D), lambda b: (b, 0)),
        ],
        out_specs=pl.BlockSpec((BN, D), lambda b: (b, 0)),
        out_shape=jax.ShapeDtypeStruct((N, D), jnp.float32),
    )(partials, x)


def kernel(x, edge_index, edge_type, edge_weight, W):
    src = edge_index[0]
    dst = edge_index[1]
    gidx = edge_type * N + src
    pad = EPAD - E
    pad_dst = N + (jnp.arange(pad, dtype=jnp.int32) % (NPAD - N))
    pad_gidx = jnp.arange(pad, dtype=jnp.int32) % 1024
    meta = jnp.concatenate([
        jnp.stack([gidx, dst]),
        jnp.stack([pad_gidx, pad_dst])], axis=1)
    meta = meta.reshape(2, CPW * NW, K).transpose(1, 0, 2)
    ew = jnp.pad(edge_weight, (0, pad)).reshape(CPW * NW, K)

    y = _relation_matmul(x, W).reshape(R * N, D)
    partials = _sc_segment_sum(y, meta, ew)
    return _epilogue(partials, x)


# speedup vs baseline: 2.5960x; 1.3164x over previous
import functools

import jax
import jax.numpy as jnp
from jax import lax
from jax.experimental import pallas as pl
from jax.experimental.pallas import tpu as pltpu
from jax.experimental.pallas import tpu_sc as plsc

N = 10000
E = 320000
D = 128
R = 8

K = 128
NC = 2
NS = 16
NW = NC * NS
CPW = 80
EPAD = NW * CPW * K
NPAD = 10240
ROWS_PER_TILE = NPAD // NS


def _ymm_body(x_ref, w_ref, y_ref):
    y_ref[0] = jnp.dot(x_ref[...], w_ref[0], preferred_element_type=jnp.float32)


def _relation_matmul(x, W):
    BN = 2000
    return pl.pallas_call(
        _ymm_body,
        grid=(R, N // BN),
        in_specs=[
            pl.BlockSpec((BN, D), lambda r, b: (b, 0)),
            pl.BlockSpec((1, D, D), lambda r, b: (r, 0, 0)),
        ],
        out_specs=pl.BlockSpec((1, BN, D), lambda r, b: (r, b, 0)),
        out_shape=jax.ShapeDtypeStruct((R, N, D), jnp.float32),
    )(x, W)


def _make_sc_segment_sum():
    mesh = plsc.VectorSubcoreMesh(core_axis_name="c", subcore_axis_name="s")

    @functools.partial(
        pl.kernel,
        out_type=jax.ShapeDtypeStruct((NC, NPAD, D), jnp.float32),
        mesh=mesh,
        scratch_types=[
            pltpu.VMEM((2, K), jnp.int32),
            pltpu.VMEM((2, K), jnp.int32),
            pltpu.VMEM((K,), jnp.float32),
            pltpu.VMEM((K,), jnp.float32),
            pltpu.VMEM((K, D), jnp.float32),
            pltpu.VMEM((K, D), jnp.float32),
            pltpu.VMEM_SHARED((NPAD, D), jnp.float32),
            pltpu.SemaphoreType.DMA,
            pltpu.SemaphoreType.DMA,
        ],
    )
    def sc_segment_sum(y_hbm, meta_hbm, ew_hbm, out_hbm,
                       meta0, meta1, ew0, ew1, rows0, rows1, acc,
                       sem_g0, sem_g1):
        cid = lax.axis_index("c")
        sid = lax.axis_index("s")
        wid = sid * NC + cid
        meta = (meta0, meta1)
        ew = (ew0, ew1)
        rows = (rows0, rows1)
        sem_g = (sem_g0, sem_g1)

        def zrow(j, _):
            for i in range(D // 16):
                rows0[j, pl.ds(i * 16, 16)] = jnp.zeros((16,), jnp.float32)
            return _
        lax.fori_loop(0, K, zrow, None)
        base = sid * ROWS_PER_TILE
        for t in range(ROWS_PER_TILE // K):
            pltpu.sync_copy(rows0, acc.at[pl.ds(base + t * K, K)])
        plsc.subcore_barrier()

        def scale(p):
            rows_v, ew_v = rows[p], ew[p]

            def row_body(jb, carry):
                ew16 = ew_v[pl.ds(jb * 16, 16)]
                for q in range(16):
                    w = ew16[q]
                    jj = jb * 16 + q
                    for i in range(D // 16):
                        rows_v[jj, pl.ds(i * 16, 16)] = (
                            rows_v[jj, pl.ds(i * 16, 16)] * w)
                return carry
            lax.fori_loop(0, K // 16, row_body, None)

        def meta_load(l, p):
            pltpu.sync_copy(meta_hbm.at[wid + l * NW], meta[p])
            pltpu.sync_copy(ew_hbm.at[wid + l * NW], ew[p])

        def gather_start(p):
            pltpu.async_copy(y_hbm.at[meta[p].at[0]], rows[p], sem_g[p])

        def gather_wait(p):
            pltpu.make_async_copy(y_hbm.at[meta[p].at[0]], rows[p],
                                  sem_g[p]).wait()

        def scatter(p):
            pltpu.sync_copy(rows[p], acc.at[meta[p].at[1]], add=True)

        meta_load(0, 0)
        gather_start(0)

        def pair_body(j2, _):
            for t in range(2):
                l = j2 * 2 + t
                p = t
                gather_wait(p)
                meta_load(l + 1, 1 - p)
                gather_start(1 - p)
                scale(p)
                scatter(p)
            return _
        lax.fori_loop(0, CPW // 2 - 1, pair_body, None)

        for t in range(2):
            l = CPW - 2 + t
            p = t
            gather_wait(p)
            if t == 0:
                meta_load(l + 1, 1 - p)
                gather_start(1 - p)
            scale(p)
            scatter(p)
        plsc.subcore_barrier()

        pltpu.sync_copy(acc.at[pl.ds(base, ROWS_PER_TILE)],
                        out_hbm.at[cid, pl.ds(base, ROWS_PER_TILE)])

    return sc_segment_sum


_sc_segment_sum = _make_sc_segment_sum()


def _epilogue_body(p_ref, x_ref, o_ref):
    m = p_ref[0] + p_ref[1]
    n1 = jnp.sqrt(jnp.sum(m * m, axis=1, keepdims=True))
    m = m / jnp.maximum(n1, 1e-6)
    h = m + x_ref[...]
    n2 = jnp.sqrt(jnp.sum(h * h, axis=1, keepdims=True))
    o_ref[...] = h / n2


def _epilogue(partials, x):
    BN = 2000
    return pl.pallas_call(
        _epilogue_body,
        grid=(N // BN,),
        in_specs=[
            pl.BlockSpec((NC, BN, D), lambda b: (0, b, 0)),
            pl.BlockSpec((BN, D), lambda b: (b, 0)),
        ],
        out_specs=pl.BlockSpec((BN, D), lambda b: (b, 0)),
        out_shape=jax.ShapeDtypeStruct((N, D), jnp.float32),
    )(partials, x)


def kernel(x, edge_index, edge_type, edge_weight, W):
    src = edge_index[0]
    dst = edge_index[1]
    gidx = edge_type * N + src
    pad = EPAD - E
    pad_dst = N + (jnp.arange(pad, dtype=jnp.int32) % (NPAD - N))
    pad_gidx = jnp.arange(pad, dtype=jnp.int32) % 1024
    meta = jnp.concatenate([
        jnp.stack([gidx, dst]),
        jnp.stack([pad_gidx, pad_dst])], axis=1)
    meta = meta.reshape(2, CPW * NW, K).transpose(1, 0, 2)
    ew = jnp.pad(edge_weight, (0, pad)).reshape(CPW * NW, K)

    y = _relation_matmul(x, W).reshape(R * N, D)
    partials = _sc_segment_sum(y, meta, ew)
    return _epilogue(partials, x)


# speedup vs baseline: 3.7457x; 1.4429x over previous
import functools

import jax
import jax.numpy as jnp
from jax import lax
from jax.experimental import pallas as pl
from jax.experimental.pallas import tpu as pltpu
from jax.experimental.pallas import tpu_sc as plsc

N = 10000
E = 320000
D = 128
R = 8

K = 128
NC = 2
NS = 16
NW = NC * NS
CPW = 80
EPAD = NW * CPW * K
NPAD = 10240
ROWS_PER_TILE = NPAD // NS


def _ymm_body(x_ref, w_ref, y_ref):
    y_ref[0] = jnp.dot(x_ref[...], w_ref[0], preferred_element_type=jnp.float32)


def _relation_matmul(x, W):
    return pl.pallas_call(
        _ymm_body,
        grid=(R,),
        in_specs=[
            pl.BlockSpec((N, D), lambda r: (0, 0)),
            pl.BlockSpec((1, D, D), lambda r: (r, 0, 0)),
        ],
        out_specs=pl.BlockSpec((1, N, D), lambda r: (r, 0, 0)),
        out_shape=jax.ShapeDtypeStruct((R, N, D), jnp.float32),
    )(x, W)


def _make_sc_segment_sum():
    mesh = plsc.VectorSubcoreMesh(core_axis_name="c", subcore_axis_name="s")

    @functools.partial(
        pl.kernel,
        out_type=jax.ShapeDtypeStruct((NC, NPAD, D), jnp.float32),
        mesh=mesh,
        scratch_types=[
            pltpu.VMEM((2, K), jnp.int32),
            pltpu.VMEM((2, K), jnp.int32),
            pltpu.VMEM((2, K), jnp.int32),
            pltpu.VMEM((2, K), jnp.int32),
            pltpu.VMEM((K,), jnp.float32),
            pltpu.VMEM((K,), jnp.float32),
            pltpu.VMEM((K,), jnp.float32),
            pltpu.VMEM((K,), jnp.float32),
            pltpu.VMEM((K, D), jnp.float32),
            pltpu.VMEM((K, D), jnp.float32),
            pltpu.VMEM_SHARED((NPAD, D), jnp.float32),
            pltpu.SemaphoreType.DMA,
            pltpu.SemaphoreType.DMA,
            pltpu.SemaphoreType.DMA,
            pltpu.SemaphoreType.DMA,
            pltpu.SemaphoreType.DMA,
            pltpu.SemaphoreType.DMA,
        ],
    )
    def sc_segment_sum(y_hbm, gidx_hbm, dst_hbm, ew_hbm, out_hbm,
                       meta0, meta1, meta2, meta3, ew0, ew1, ew2, ew3,
                       rows0, rows1, acc,
                       sem_g0, sem_g1, sem_m0, sem_m1, sem_m2, sem_m3):
        cid = lax.axis_index("c")
        sid = lax.axis_index("s")
        wid = sid * NC + cid
        meta = (meta0, meta1, meta2, meta3)
        ew = (ew0, ew1, ew2, ew3)
        rows = (rows0, rows1)
        sem_g = (sem_g0, sem_g1)
        sem_m = (sem_m0, sem_m1, sem_m2, sem_m3)

        def zrow(j, _):
            for i in range(D // 16):
                rows0[j, pl.ds(i * 16, 16)] = jnp.zeros((16,), jnp.float32)
            return _
        lax.fori_loop(0, K, zrow, None)
        base = sid * ROWS_PER_TILE
        for t in range(ROWS_PER_TILE // K):
            pltpu.sync_copy(rows0, acc.at[pl.ds(base + t * K, K)])
        plsc.subcore_barrier()

        def scale(p, m):
            rows_v, ew_v = rows[p], ew[m]

            @plsc.parallel_loop(0, K // 16)
            def row_body(jb):
                ew16 = ew_v[pl.ds(jb * 16, 16)]
                for q in range(16):
                    w = ew16[q]
                    jj = jb * 16 + q
                    for i in range(D // 16):
                        rows_v[jj, pl.ds(i * 16, 16)] = (
                            rows_v[jj, pl.ds(i * 16, 16)] * w)

        def meta_start(l, m):
            c = wid + l * NW
            pltpu.async_copy(gidx_hbm.at[c], meta[m].at[0], sem_m[m])
            pltpu.async_copy(dst_hbm.at[c], meta[m].at[1], sem_m[m])
            pltpu.async_copy(ew_hbm.at[c], ew[m], sem_m[m])

        def meta_wait(l, m):
            c = wid + l * NW
            pltpu.make_async_copy(gidx_hbm.at[c], meta[m].at[0],
                                  sem_m[m]).wait()
            pltpu.make_async_copy(dst_hbm.at[c], meta[m].at[1],
                                  sem_m[m]).wait()
            pltpu.make_async_copy(ew_hbm.at[c], ew[m], sem_m[m]).wait()

        def gather_start(p, m):
            pltpu.async_copy(y_hbm.at[meta[m].at[0]], rows[p], sem_g[p])

        def gather_wait(p, m):
            pltpu.make_async_copy(y_hbm.at[meta[m].at[0]], rows[p],
                                  sem_g[p]).wait()

        def scatter(p, m):
            pltpu.sync_copy(rows[p], acc.at[meta[m].at[1]], add=True)

        meta_start(0, 0)
        meta_wait(0, 0)
        gather_start(0, 0)
        meta_start(1, 1)

        def quad_body(j4, _):
            for t in range(4):
                l = j4 * 4 + t
                p = t % 2
                gather_wait(p, t)
                meta_wait(l + 1, (t + 1) % 4)
                gather_start(1 - p, (t + 1) % 4)
                meta_start(l + 2, (t + 2) % 4)
                scale(p, t)
                scatter(p, t)
            return _
        lax.fori_loop(0, CPW // 4 - 1, quad_body, None)

        for t in range(4):
            l = CPW - 4 + t
            p = t % 2
            gather_wait(p, t)
            if l + 1 < CPW:
                meta_wait(l + 1, (t + 1) % 4)
                gather_start(1 - p, (t + 1) % 4)
            if l + 2 < CPW:
                meta_start(l + 2, (t + 2) % 4)
            scale(p, t)
            scatter(p, t)
        plsc.subcore_barrier()

        pltpu.sync_copy(acc.at[pl.ds(base, ROWS_PER_TILE)],
                        out_hbm.at[cid, pl.ds(base, ROWS_PER_TILE)])

    return sc_segment_sum


_sc_segment_sum = _make_sc_segment_sum()


def _epilogue_body(p_ref, x_ref, o_ref):
    m = p_ref[0] + p_ref[1]
    n1 = jnp.sqrt(jnp.sum(m * m, axis=1, keepdims=True))
    m = m / jnp.maximum(n1, 1e-6)
    h = m + x_ref[...]
    n2 = jnp.sqrt(jnp.sum(h * h, axis=1, keepdims=True))
    o_ref[...] = h / n2


def _epilogue(partials, x):
    BN = 2000
    return pl.pallas_call(
        _epilogue_body,
        grid=(N // BN,),
        in_specs=[
            pl.BlockSpec((NC, BN, D), lambda b: (0, b, 0)),
            pl.BlockSpec((BN, D), lambda b: (b, 0)),
        ],
        out_specs=pl.BlockSpec((BN, D), lambda b: (b, 0)),
        out_shape=jax.ShapeDtypeStruct((N, D), jnp.float32),
    )(partials, x)


def kernel(x, edge_index, edge_type, edge_weight, W):
    src = edge_index[0]
    dst = edge_index[1]
    gidx = edge_type * N + src
    nchunk = E // K
    padc = CPW * NW - nchunk
    pad_e = jnp.arange(padc * K, dtype=jnp.int32)
    gidx_a = jnp.concatenate([gidx.reshape(nchunk, K),
                              (pad_e % 1024).reshape(padc, K)], axis=0)
    dst_a = jnp.concatenate([dst.reshape(nchunk, K),
                             (N + pad_e % (NPAD - N)).reshape(padc, K)],
                            axis=0)
    ew_a = jnp.concatenate([edge_weight.reshape(nchunk, K),
                            jnp.zeros((padc, K), jnp.float32)], axis=0)

    y = _relation_matmul(x, W).reshape(R * N, D)
    partials = _sc_segment_sum(y, gidx_a, dst_a, ew_a)
    return _epilogue(partials, x)
